# initial kernel scaffold (unmeasured)
import jax
import jax.numpy as jnp
from jax import lax
from jax.experimental import pallas as pl
from jax.experimental.pallas import tpu as pltpu

N_LAYERS = 3
STAGE_MASKS = (1, 3, 4)
N_EXCH = N_LAYERS * len(STAGE_MASKS)


def kernel(x, Win0, Wout0, Win1, Wout1, Win2, Wout2):
    b, d_sh = x.shape
    _, h_dim = Win0.shape
    _, o_sh = Wout0.shape

    def body(x_ref, win0_ref, wout0_ref, win1_ref, wout1_ref, win2_ref,
             wout2_ref, out_ref, h_ref, rbuf_ref, send_sems, recv_sems):
        my_p = lax.axis_index("i")
        wins = (win0_ref, win1_ref, win2_ref)
        wouts = (wout0_ref, wout1_ref, wout2_ref)

        xv = x_ref[...].astype(jnp.bfloat16)
        for layer in range(N_LAYERS):
            w_in = wins[layer][...].astype(jnp.bfloat16)
            partial = jnp.dot(xv, w_in, preferred_element_type=jnp.float32)
            h_ref[...] = partial.astype(jnp.bfloat16)

            for s, mask in enumerate(STAGE_MASKS):
                k = layer * len(STAGE_MASKS) + s
                partner = jnp.bitwise_xor(my_p, mask)
                rdma = pltpu.make_async_remote_copy(
                    src_ref=h_ref,
                    dst_ref=rbuf_ref.at[k],
                    send_sem=send_sems.at[k],
                    recv_sem=recv_sems.at[k],
                    device_id=(partner,),
                    device_id_type=pl.DeviceIdType.MESH,
                )
                rdma.start()
                rdma.wait()
                h_ref[...] = h_ref[...] + rbuf_ref[k]

            hv = jnp.maximum(h_ref[...], 0).astype(jnp.bfloat16)
            w_out = wouts[layer][...].astype(jnp.bfloat16)
            if layer < N_LAYERS - 1:
                xv = jnp.dot(
                    hv, w_out, preferred_element_type=jnp.float32
                ).astype(jnp.bfloat16)
            else:
                out_ref[...] = jnp.dot(
                    hv, w_out, preferred_element_type=jnp.float32
                )

    return pl.pallas_call(
        body,
        out_shape=jax.ShapeDtypeStruct((b, o_sh), jnp.float32),
        in_specs=[pl.BlockSpec(memory_space=pltpu.VMEM)] * 7,
        out_specs=pl.BlockSpec(memory_space=pltpu.VMEM),
        scratch_shapes=[
            pltpu.VMEM((b, h_dim), jnp.bfloat16),
            pltpu.VMEM((N_EXCH, b, h_dim), jnp.bfloat16),
            pltpu.SemaphoreType.DMA((N_EXCH,)),
            pltpu.SemaphoreType.DMA((N_EXCH,)),
        ],
        compiler_params=pltpu.CompilerParams(collective_id=0),
    )(x, Win0, Wout0, Win1, Wout1, Win2, Wout2)


# baseline (device time: 84645 ns/iter reference)
import jax
import jax.numpy as jnp
from jax import lax
from jax.experimental import pallas as pl
from jax.experimental.pallas import tpu as pltpu

N_LAYERS = 3
STAGE_MASKS = (1, 3, 4)
N_EXCH = N_LAYERS * len(STAGE_MASKS)


def kernel(x, Win0, Wout0, Win1, Wout1, Win2, Wout2):
    b, d_sh = x.shape
    _, h_dim = Win0.shape
    _, o_sh = Wout0.shape

    def body(x_ref, win0_ref, wout0_ref, win1_ref, wout1_ref, win2_ref,
             wout2_ref, out_ref, h_ref, rbuf_ref, send_sems, recv_sems):
        my_p = lax.axis_index("i")
        wins = (win0_ref, win1_ref, win2_ref)
        wouts = (wout0_ref, wout1_ref, wout2_ref)

        xv = x_ref[...].astype(jnp.bfloat16)
        for layer in range(N_LAYERS):
            w_in = wins[layer][...].astype(jnp.bfloat16)
            partial = jnp.dot(xv, w_in, preferred_element_type=jnp.float32)
            h_ref[...] = partial.astype(jnp.bfloat16)

            for s, mask in enumerate(STAGE_MASKS):
                k = layer * len(STAGE_MASKS) + s
                partner = jnp.bitwise_xor(my_p, mask)
                rdma = pltpu.make_async_remote_copy(
                    src_ref=h_ref,
                    dst_ref=rbuf_ref.at[k],
                    send_sem=send_sems.at[k],
                    recv_sem=recv_sems.at[k],
                    device_id=(partner,),
                    device_id_type=pl.DeviceIdType.MESH,
                )
                rdma.start()
                rdma.wait()
                h_ref[...] = h_ref[...] + rbuf_ref[k]

            hv = jnp.maximum(h_ref[...], 0).astype(jnp.bfloat16)
            w_out = wouts[layer][...].astype(jnp.bfloat16)
            if layer < N_LAYERS - 1:
                xv = jnp.dot(
                    hv, w_out, preferred_element_type=jnp.float32
                ).astype(jnp.bfloat16)
            else:
                out_ref[...] = jnp.dot(
                    hv, w_out, preferred_element_type=jnp.float32
                )

    return pl.pallas_call(
        body,
        out_shape=jax.ShapeDtypeStruct((b, o_sh), jnp.float32),
        in_specs=[pl.BlockSpec(memory_space=pltpu.VMEM)] * 7,
        out_specs=pl.BlockSpec(memory_space=pltpu.VMEM),
        scratch_shapes=[
            pltpu.VMEM((b, h_dim), jnp.bfloat16),
            pltpu.VMEM((N_EXCH, b, h_dim), jnp.bfloat16),
            pltpu.SemaphoreType.DMA((N_EXCH,)),
            pltpu.SemaphoreType.DMA((N_EXCH,)),
        ],
    )(x, Win0, Wout0, Win1, Wout1, Win2, Wout2)


# device time: 49299 ns/iter; 1.7170x vs baseline; 1.7170x over previous
import jax
import jax.numpy as jnp
from jax import lax
from jax.experimental import pallas as pl
from jax.experimental.pallas import tpu as pltpu

N_DEV = 8
N_LAYERS = 3
CHUNK = 64

P_OF_Q = (0, 1, 3, 2, 4, 5, 7, 6)


def kernel(x, Win0, Wout0, Win1, Wout1, Win2, Wout2):
    b, d_sh = x.shape
    _, h_dim = Win0.shape
    _, o_sh = Wout0.shape

    def body(x_ref, win0_ref, wout0_ref, win1_ref, wout1_ref, win2_ref,
             wout2_ref, out_ref, h_ref, rs_rbuf, hg_ref, xn_ref,
             rs_send_sems, rs_recv_sems, ag_send_sems, ag_recv_sems):
        my_p = lax.axis_index("i")
        p0 = jnp.bitwise_and(my_p, 1)
        p1 = jnp.bitwise_and(my_p >> 1, 1)
        p2 = jnp.bitwise_and(my_p >> 2, 1)
        my_q = 4 * p2 + 2 * p1 + jnp.bitwise_xor(p0, p1)
        my_row = my_q * CHUNK

        wins = (win0_ref, win1_ref, win2_ref)
        wouts = (wout0_ref, wout1_ref, wout2_ref)

        xv = x_ref[...].astype(jnp.bfloat16)
        for layer in range(N_LAYERS):
            w_in = wins[layer][...].astype(jnp.bfloat16)
            partial = jnp.dot(xv, w_in, preferred_element_type=jnp.float32)
            h_ref[...] = partial.astype(jnp.bfloat16)

            for c in range(N_DEV):
                dest = P_OF_Q[c]
                k = layer * N_DEV + dest

                @pl.when(my_q != c)
                def _(c=c, dest=dest, k=k):
                    rdma = pltpu.make_async_remote_copy(
                        src_ref=h_ref.at[pl.ds(c * CHUNK, CHUNK)],
                        dst_ref=rs_rbuf.at[my_p],
                        send_sem=rs_send_sems.at[k],
                        recv_sem=rs_recv_sems.at[k - dest + my_p],
                        device_id=(dest,),
                        device_id_type=pl.DeviceIdType.MESH,
                    )
                    rdma.start()

            acc = h_ref[pl.ds(my_row, CHUNK), :].astype(jnp.float32)
            for j in range(N_DEV):
                k = layer * N_DEV + j

                @pl.when(my_p != j)
                def _(j=j, k=k):
                    recv = pltpu.make_async_remote_copy(
                        src_ref=rs_rbuf.at[j],
                        dst_ref=rs_rbuf.at[j],
                        send_sem=rs_send_sems.at[k],
                        recv_sem=rs_recv_sems.at[k],
                        device_id=(j,),
                        device_id_type=pl.DeviceIdType.MESH,
                    )
                    recv.wait_recv()

                acc = acc + jnp.where(
                    my_p != j, rs_rbuf[j].astype(jnp.float32), 0.0
                )

            myh = jnp.maximum(acc, 0.0).astype(jnp.bfloat16)
            hg_ref[pl.ds(my_row, CHUNK), :] = myh

            for j in range(N_DEV):
                k = layer * N_DEV + j

                @pl.when(my_p != j)
                def _(j=j, k=k):
                    rdma = pltpu.make_async_remote_copy(
                        src_ref=hg_ref.at[pl.ds(my_row, CHUNK)],
                        dst_ref=hg_ref.at[pl.ds(my_row, CHUNK)],
                        send_sem=ag_send_sems.at[k],
                        recv_sem=ag_recv_sems.at[layer * N_DEV + my_p],
                        device_id=(j,),
                        device_id_type=pl.DeviceIdType.MESH,
                    )
                    rdma.start()

            for j in range(N_DEV):
                k = layer * N_DEV + j

                @pl.when(my_p != j)
                def _(j=j, k=k):
                    qj = P_OF_Q[j]
                    recv = pltpu.make_async_remote_copy(
                        src_ref=hg_ref.at[pl.ds(qj * CHUNK, CHUNK)],
                        dst_ref=hg_ref.at[pl.ds(qj * CHUNK, CHUNK)],
                        send_sem=ag_send_sems.at[k],
                        recv_sem=ag_recv_sems.at[k],
                        device_id=(j,),
                        device_id_type=pl.DeviceIdType.MESH,
                    )
                    recv.wait_recv()

            hv = hg_ref[...]
            w_out = wouts[layer][...].astype(jnp.bfloat16)
            if layer < N_LAYERS - 1:
                xv = jnp.dot(
                    hv, w_out, preferred_element_type=jnp.float32
                ).astype(jnp.bfloat16)
            else:
                out_ref[...] = jnp.dot(
                    hv, w_out, preferred_element_type=jnp.float32
                )

            for j in range(N_DEV):
                k = layer * N_DEV + j

                @pl.when(my_p != j)
                def _(j=j, k=k):
                    for sems in (rs_send_sems, ag_send_sems):
                        w = pltpu.make_async_remote_copy(
                            src_ref=rs_rbuf.at[j],
                            dst_ref=rs_rbuf.at[j],
                            send_sem=sems.at[k],
                            recv_sem=rs_recv_sems.at[k],
                            device_id=(j,),
                            device_id_type=pl.DeviceIdType.MESH,
                        )
                        w.wait_send()

    n_sems = N_LAYERS * N_DEV
    return pl.pallas_call(
        body,
        out_shape=jax.ShapeDtypeStruct((b, o_sh), jnp.float32),
        in_specs=[pl.BlockSpec(memory_space=pltpu.VMEM)] * 7,
        out_specs=pl.BlockSpec(memory_space=pltpu.VMEM),
        scratch_shapes=[
            pltpu.VMEM((b, h_dim), jnp.bfloat16),
            pltpu.VMEM((N_DEV, CHUNK, h_dim), jnp.bfloat16),
            pltpu.VMEM((b, h_dim), jnp.bfloat16),
            pltpu.VMEM((b, o_sh), jnp.bfloat16),
            pltpu.SemaphoreType.DMA((n_sems,)),
            pltpu.SemaphoreType.DMA((n_sems,)),
            pltpu.SemaphoreType.DMA((n_sems,)),
            pltpu.SemaphoreType.DMA((n_sems,)),
        ],
    )(x, Win0, Wout0, Win1, Wout1, Win2, Wout2)
